# baseline (speedup 1.0000x reference)
import jax
import jax.numpy as jnp
from jax.experimental import pallas as pl

_N, _T, _P, _D = 8, 48, 325, 512
_M = _N * _T * _P
_BM = 2600


def _body(x_ref, m_ref, tw_ref, c_ref, vw_ref, et_ref, ut_ref, out_ref):
    xv = x_ref[...]
    v = xv[:, 0:1]
    t = xv[:, 1:2]
    mon = m_ref[...] > 0.5
    bad = jnp.isnan(v)
    ve = jnp.where(bad, et_ref[...], v * vw_ref[...])
    ve = jnp.where(mon, ve, ut_ref[...])
    out_ref[...] = t * tw_ref[...] + c_ref[...] + ve


def kernel(x, monitor_mask, time_emb_w, time_emb_b, value_emb_w, value_emb_b,
           empty_token, unmonitored_token):
    xf = x.reshape(_M, 2)
    ms = monitor_mask.astype(jnp.float32).reshape(_M, 1)
    c = time_emb_b + value_emb_b
    et = empty_token.reshape(1, _D) - value_emb_b
    ut = unmonitored_token.reshape(1, _D) - value_emb_b

    out = pl.pallas_call(
        _body,
        grid=(_M // _BM,),
        in_specs=[pl.BlockSpec((_BM, 2), lambda i: (i, 0)),
                  pl.BlockSpec((_BM, 1), lambda i: (i, 0)),
                  pl.BlockSpec((1, _D), lambda i: (0, 0)),
                  pl.BlockSpec((1, _D), lambda i: (0, 0)),
                  pl.BlockSpec((1, _D), lambda i: (0, 0)),
                  pl.BlockSpec((1, _D), lambda i: (0, 0)),
                  pl.BlockSpec((1, _D), lambda i: (0, 0))],
        out_specs=pl.BlockSpec((_BM, _D), lambda i: (i, 0)),
        out_shape=jax.ShapeDtypeStruct((_M, _D), jnp.float32),
    )(xf, ms, time_emb_w, c, value_emb_w, et, ut)
    return out.reshape(_N, _T, _P, _D)
